# Initial kernel scaffold; baseline (speedup 1.0000x reference)
import jax
import jax.numpy as jnp
from jax.experimental import pallas as pl


def kernel(pr, track_pad_mask, W_cnn, b_cnn, codebook, W_fc, b_fc, W_mu, b_mu):
    raise NotImplementedError("write your pallas kernel here")


# speedup vs baseline: 7.1977x; 7.1977x over previous
import functools

import jax
import jax.numpy as jnp
from jax.experimental import pallas as pl
from jax.experimental.pallas import tpu as pltpu

BS = 16384
L = 32
NC = 10
ZD = 128
K = 128
T = 8
D = NC

CHUNK = 2048
NSTEPS = BS // CHUNK


def _body(pr_ref, valid_ref, wc_ref, bcnn_ref, cbT_ref, c2_ref, ctT_ref,
          bcomb_ref, out_ref, cmt_ref, perp_ref, counts_ref, acc_ref):
    i = pl.program_id(0)

    @pl.when(i == 0)
    def _init():
        counts_ref[...] = jnp.zeros_like(counts_ref)
        acc_ref[...] = jnp.zeros_like(acc_ref)

    pr = pr_ref[...]
    valid = valid_ref[...]
    out_acc = jnp.broadcast_to(bcomb_ref[...], (CHUNK, ZD))
    counts = counts_ref[...]
    loss = jnp.zeros((1, 1), jnp.float32)

    for t in range(T):
        pt = pr[:, 4 * t:4 * t + 4]
        zt = jnp.dot(pt, wc_ref[...], preferred_element_type=jnp.float32)
        zt = jnp.maximum(zt + bcnn_ref[...], 0.0)
        scores = jnp.dot(zt, cbT_ref[...], preferred_element_type=jnp.float32)
        z2 = jnp.sum(zt * zt, axis=1, keepdims=True)
        d2 = z2 - 2.0 * scores + c2_ref[...]
        mind = jnp.min(d2, axis=1, keepdims=True)
        amin = jnp.argmin(d2, axis=1).astype(jnp.int32)
        enc = (jax.lax.broadcasted_iota(jnp.int32, (CHUNK, K), 1)
               == amin[:, None]).astype(jnp.float32)
        out_acc = out_acc + jnp.dot(enc, ctT_ref[t],
                                    preferred_element_type=jnp.float32)
        counts = counts + jnp.sum(enc * valid, axis=0, keepdims=True)
        loss = loss + jnp.sum(mind * valid).reshape(1, 1)

    out_ref[...] = out_acc
    counts_ref[...] = counts
    vsum = jnp.sum(valid).reshape(1, 1)
    acc = acc_ref[...]
    acc_ref[...] = acc + jnp.concatenate([loss, vsum], axis=1)

    @pl.when(i == NSTEPS - 1)
    def _fin():
        a = acc_ref[...]
        loss_sum = a[:, 0:1]
        n8 = a[:, 1:2] * T
        e_latent = loss_sum / (n8 * D + 1e-9)
        cmt_ref[...] = 0.25 * e_latent
        p = counts_ref[...] / (n8 + 1e-9)
        ent = -jnp.sum(p * jnp.log(p + 1e-10), axis=1, keepdims=True)
        perp_ref[...] = jnp.exp(ent)


@functools.partial(jax.jit, static_argnames=())
def kernel(pr, track_pad_mask, W_cnn, b_cnn, codebook, W_fc, b_fc, W_mu, b_mu):
    W_comb = W_mu @ W_fc
    b_comb = (W_mu @ b_fc + b_mu)[None, :]
    Wr = W_comb.reshape(ZD, NC, T)
    CtT = jnp.einsum('kc,zct->tkz', codebook, Wr)
    Wc = W_cnn[:, 0, :].T
    bcnn = b_cnn[None, :]
    cbT = codebook.T
    c2 = jnp.sum(codebook * codebook, axis=1)[None, :]
    validf = 1.0 - track_pad_mask.astype(jnp.float32)

    out, cmt, perp = pl.pallas_call(
        _body,
        grid=(NSTEPS,),
        in_specs=[
            pl.BlockSpec((CHUNK, L), lambda i: (i, 0)),
            pl.BlockSpec((CHUNK, 1), lambda i: (i, 0)),
            pl.BlockSpec((4, NC), lambda i: (0, 0)),
            pl.BlockSpec((1, NC), lambda i: (0, 0)),
            pl.BlockSpec((NC, K), lambda i: (0, 0)),
            pl.BlockSpec((1, K), lambda i: (0, 0)),
            pl.BlockSpec((T, K, ZD), lambda i: (0, 0, 0)),
            pl.BlockSpec((1, ZD), lambda i: (0, 0)),
        ],
        out_specs=[
            pl.BlockSpec((CHUNK, ZD), lambda i: (i, 0)),
            pl.BlockSpec((1, 1), lambda i: (0, 0)),
            pl.BlockSpec((1, 1), lambda i: (0, 0)),
        ],
        out_shape=[
            jax.ShapeDtypeStruct((BS, ZD), jnp.float32),
            jax.ShapeDtypeStruct((1, 1), jnp.float32),
            jax.ShapeDtypeStruct((1, 1), jnp.float32),
        ],
        scratch_shapes=[
            pltpu.VMEM((1, K), jnp.float32),
            pltpu.VMEM((1, 2), jnp.float32),
        ],
    )(pr, validf, Wc, bcnn, cbT, c2, CtT, b_comb)

    return (out, cmt[0, 0], perp[0, 0])


# speedup vs baseline: 9.9114x; 1.3770x over previous
import functools

import jax
import jax.numpy as jnp
from jax.experimental import pallas as pl
from jax.experimental.pallas import tpu as pltpu

BS = 16384
L = 32
NC = 10
ZD = 128
K = 128
T = 8
D = NC

CHUNK = 2048
NSTEPS = BS // CHUNK

_ROWC = (((0,), (0,)), ((), ()))


def _body(pr_ref, valid_ref, wbig_ref, bcnn_ref, mbig_ref, ctall_ref,
          bcomb_ref, out_ref, cmt_ref, perp_ref, counts_ref, acc_ref):
    i = pl.program_id(0)

    @pl.when(i == 0)
    def _init():
        counts_ref[...] = jnp.zeros_like(counts_ref)
        acc_ref[...] = jnp.zeros_like(acc_ref)

    pr = pr_ref[...]
    valid = valid_ref[...]

    z_all = jnp.dot(pr, wbig_ref[...], preferred_element_type=jnp.float32)
    z_all = jnp.maximum(z_all + bcnn_ref[...], 0.0)

    zta = jnp.concatenate(
        [z_all, jnp.ones((CHUNK, 1), jnp.float32)], axis=1)
    s_all = jnp.dot(zta, mbig_ref[...],
                    preferred_element_type=jnp.float32)

    iota = jax.lax.broadcasted_iota(jnp.int32, (CHUNK, K), 1)
    encs = []
    dmin_sum = jnp.zeros((CHUNK, 1), jnp.float32)
    for t in range(T):
        s_t = s_all[:, t * K:(t + 1) * K]
        dmin_sum = dmin_sum + jnp.min(s_t, axis=1, keepdims=True)
        amin = jnp.argmin(s_t, axis=1).astype(jnp.int32)
        encs.append((iota == amin[:, None]).astype(jnp.float32))
    enc_all = jnp.concatenate(encs, axis=1)

    out_ref[...] = bcomb_ref[...] + jnp.dot(
        enc_all, ctall_ref[...], preferred_element_type=jnp.float32)

    counts_ref[...] = counts_ref[...] + jax.lax.dot_general(
        valid, enc_all, _ROWC, preferred_element_type=jnp.float32)
    z2sum = jnp.sum(z_all * z_all, axis=1, keepdims=True)
    loss = jax.lax.dot_general(valid, dmin_sum + z2sum, _ROWC,
                               preferred_element_type=jnp.float32)
    vsum = jnp.sum(valid).reshape(1, 1)
    acc_ref[...] = acc_ref[...] + jnp.concatenate([loss, vsum], axis=1)

    @pl.when(i == NSTEPS - 1)
    def _fin():
        a = acc_ref[...]
        loss_sum = a[:, 0:1]
        n8 = a[:, 1:2] * T
        e_latent = loss_sum / (n8 * D + 1e-9)
        cmt_ref[...] = 0.25 * e_latent
        call = counts_ref[...]
        c128 = call[:, 0:K]
        for t in range(1, T):
            c128 = c128 + call[:, t * K:(t + 1) * K]
        p = c128 / (n8 + 1e-9)
        ent = -jnp.sum(p * jnp.log(p + 1e-10), axis=1, keepdims=True)
        perp_ref[...] = jnp.exp(ent)


@functools.partial(jax.jit, static_argnames=())
def kernel(pr, track_pad_mask, W_cnn, b_cnn, codebook, W_fc, b_fc, W_mu, b_mu):
    W_comb = W_mu @ W_fc
    b_comb = (W_mu @ b_fc + b_mu)[None, :]
    Wr = W_comb.reshape(ZD, NC, T)
    CtAll = jnp.einsum('kc,zct->tkz', codebook, Wr).reshape(T * K, ZD)
    Wc = W_cnn[:, 0, :].T
    Wbig = jnp.kron(jnp.eye(T, dtype=jnp.float32), Wc)
    bcnn = jnp.tile(b_cnn, T)[None, :]
    c2 = jnp.sum(codebook * codebook, axis=1)
    Mbig = jnp.concatenate([
        jnp.kron(jnp.eye(T, dtype=jnp.float32), -2.0 * codebook.T),
        jnp.tile(c2, T)[None, :],
    ], axis=0)
    validf = 1.0 - track_pad_mask.astype(jnp.float32)

    out, cmt, perp = pl.pallas_call(
        _body,
        grid=(NSTEPS,),
        in_specs=[
            pl.BlockSpec((CHUNK, L), lambda i: (i, 0)),
            pl.BlockSpec((CHUNK, 1), lambda i: (i, 0)),
            pl.BlockSpec((L, NC * T), lambda i: (0, 0)),
            pl.BlockSpec((1, NC * T), lambda i: (0, 0)),
            pl.BlockSpec((NC * T + 1, T * K), lambda i: (0, 0)),
            pl.BlockSpec((T * K, ZD), lambda i: (0, 0)),
            pl.BlockSpec((1, ZD), lambda i: (0, 0)),
        ],
        out_specs=[
            pl.BlockSpec((CHUNK, ZD), lambda i: (i, 0)),
            pl.BlockSpec((1, 1), lambda i: (0, 0)),
            pl.BlockSpec((1, 1), lambda i: (0, 0)),
        ],
        out_shape=[
            jax.ShapeDtypeStruct((BS, ZD), jnp.float32),
            jax.ShapeDtypeStruct((1, 1), jnp.float32),
            jax.ShapeDtypeStruct((1, 1), jnp.float32),
        ],
        scratch_shapes=[
            pltpu.VMEM((1, T * K), jnp.float32),
            pltpu.VMEM((1, 2), jnp.float32),
        ],
    )(pr, validf, Wbig, bcnn, Mbig, CtAll, b_comb)

    return (out, cmt[0, 0], perp[0, 0])
